# baseline (speedup 1.0000x reference)
import jax
import jax.numpy as jnp
from jax import lax
from jax.experimental import pallas as pl
from jax.experimental.pallas import tpu as pltpu
from jax.experimental.pallas import tpu_sc as plsc

VOCAB = 1000000
D = 64
PMAX = 200
B = 4096
S = 200
NC, NS = 2, 16
NW = NC * NS
SLABS_W = B // NW
SLAB_CHUNK = 2
NCHUNK = SLABS_W // SLAB_CHUNK
KSUB = 2
W_IDX = S // KSUB


def _body(seq_hbm, pid_hbm, tok_hbm, pos_hbm, out_hbm,
          idx_v, pidx_v, buf, pos_sp, sem_t, sem_p):
    c = lax.axis_index("c")
    s = lax.axis_index("s")
    wid = s * NC + c

    @pl.when(s == 0)
    def _stage():
        pltpu.sync_copy(pos_hbm, pos_sp)

    plsc.subcore_barrier()

    def chunk_body(ci, carry):
        b0 = wid * SLABS_W + ci * SLAB_CHUNK
        pltpu.sync_copy(seq_hbm.at[pl.ds(b0, SLAB_CHUNK)], idx_v)
        pltpu.sync_copy(pid_hbm.at[pl.ds(b0, SLAB_CHUNK)], pidx_v)
        for j in range(SLAB_CHUNK):
            for k in range(KSUB):
                pltpu.async_copy(pos_sp.at[pidx_v.at[j, k]],
                                 buf.at[j, pl.ds(k * W_IDX, W_IDX)], sem_p)
        for j in range(SLAB_CHUNK):
            for k in range(KSUB):
                pltpu.make_async_copy(pos_sp.at[pidx_v.at[j, k]],
                                      buf.at[j, pl.ds(k * W_IDX, W_IDX)],
                                      sem_p).wait()
        for j in range(SLAB_CHUNK):
            for k in range(KSUB):
                pltpu.async_copy(tok_hbm.at[idx_v.at[j, k]],
                                 buf.at[j, pl.ds(k * W_IDX, W_IDX)],
                                 sem_t, add=True)
        for j in range(SLAB_CHUNK):
            for k in range(KSUB):
                pltpu.make_async_copy(tok_hbm.at[idx_v.at[j, k]],
                                      buf.at[j, pl.ds(k * W_IDX, W_IDX)],
                                      sem_t).wait()
        pltpu.sync_copy(buf, out_hbm.at[pl.ds(b0, SLAB_CHUNK)])
        return carry

    lax.fori_loop(0, NCHUNK, chunk_body, 0, unroll=False)


@jax.jit
def _embed_sum(seq3, pid3, token_table, pos_table):
    mesh = plsc.VectorSubcoreMesh(core_axis_name="c", subcore_axis_name="s")
    kern = pl.kernel(
        _body,
        out_type=jax.ShapeDtypeStruct((B, S, D), jnp.float32),
        mesh=mesh,
        scratch_types=[
            pltpu.VMEM((SLAB_CHUNK, KSUB, W_IDX), jnp.int32),
            pltpu.VMEM((SLAB_CHUNK, KSUB, W_IDX), jnp.int32),
            pltpu.VMEM((SLAB_CHUNK, S, D), jnp.float32),
            pltpu.VMEM_SHARED((PMAX, D), jnp.float32),
            pltpu.SemaphoreType.DMA,
            pltpu.SemaphoreType.DMA,
        ],
        compiler_params=pltpu.CompilerParams(use_tc_tiling_on_sc=False),
    )
    return kern(seq3, pid3, token_table, pos_table)


def kernel(sequence, position_ids, token_table, pos_table):
    seq3 = sequence.reshape(B, KSUB, W_IDX).astype(jnp.int32)
    pid3 = position_ids.reshape(B, KSUB, W_IDX).astype(jnp.int32)
    return _embed_sum(seq3, pid3, token_table, pos_table)
